# Initial kernel scaffold; baseline (speedup 1.0000x reference)
import jax
import jax.numpy as jnp
from jax.experimental import pallas as pl


def kernel(h, edge_index, W1, b1, W2, b2, Wlin, blin):
    raise NotImplementedError("write your pallas kernel here")


# speedup vs baseline: 11.4934x; 11.4934x over previous
import functools

import jax
import jax.numpy as jnp
from jax import lax
from jax.experimental import pallas as pl
from jax.experimental.pallas import tpu as pltpu
from jax.experimental.pallas import tpu_sc as plsc

N = 10000
E = 320000
D = 128
NC = 2
NS = 16
NW = NC * NS
EPW = E // NW
CH = 80
NCHUNK = EPW // CH
RB = 640
RBL = N - (NS - 1) * RB

_mesh = plsc.VectorSubcoreMesh(
    core_axis_name="c", subcore_axis_name="s", num_cores=NC, num_subcores=NS)


def _zero_vec(ref, n):
    for q in range(n // 16):
        ref[pl.ds(q * 16, 16)] = jnp.zeros((16,), jnp.float32)


def _zero_rows(ref, r, w):
    for i in range(r):
        for q in range(w // 16):
            ref[i, pl.ds(q * 16, 16)] = jnp.zeros((16,), jnp.float32)


def _split_rows(sid, fn):
    @pl.when(sid < NS - 1)
    def _main():
        fn(pl.multiple_of(sid * RB, 8), RB)

    @pl.when(sid == NS - 1)
    def _last():
        fn((NS - 1) * RB, RBL)


@functools.partial(
    pl.kernel,
    out_type=[
        jax.ShapeDtypeStruct((NC * N,), jnp.float32),
        jax.ShapeDtypeStruct((NC * N,), jnp.float32),
    ],
    mesh=_mesh,
    scratch_types=[
        pltpu.VMEM((NCHUNK, CH), jnp.int32),
        pltpu.VMEM((NCHUNK, CH), jnp.int32),
        pltpu.VMEM((CH,), jnp.float32),
        pltpu.VMEM((RB,), jnp.float32),
        pltpu.VMEM_SHARED((N,), jnp.float32),
        pltpu.VMEM_SHARED((N,), jnp.float32),
    ],
)
def _sc_degrees(src_hbm, dst_hbm, dout_hbm, din_hbm,
                sidx, didx, ones_v, bounce, do_sh, di_sh):
    cid = lax.axis_index("c")
    sid = lax.axis_index("s")
    wid = cid * NS + sid
    for q in range(CH // 16):
        ones_v[pl.ds(q * 16, 16)] = jnp.ones((16,), jnp.float32)
    _zero_vec(bounce, RB)
    _split_rows(sid, lambda s, l: pltpu.sync_copy(
        bounce.at[pl.ds(0, l)], do_sh.at[pl.ds(s, l)]))
    _split_rows(sid, lambda s, l: pltpu.sync_copy(
        bounce.at[pl.ds(0, l)], di_sh.at[pl.ds(s, l)]))
    pltpu.sync_copy(src_hbm.at[wid], sidx)
    pltpu.sync_copy(dst_hbm.at[wid], didx)
    plsc.subcore_barrier()

    def body(j, carry):
        pltpu.sync_copy(ones_v, do_sh.at[sidx.at[j]], add=True)
        pltpu.sync_copy(ones_v, di_sh.at[didx.at[j]], add=True)
        return carry

    lax.fori_loop(0, NCHUNK, body, 0)
    plsc.subcore_barrier()

    def wb(sh, out):
        def cp(s, l):
            pltpu.sync_copy(sh.at[pl.ds(s, l)], bounce.at[pl.ds(0, l)])
            pltpu.sync_copy(bounce.at[pl.ds(0, l)],
                            out.at[pl.ds(pl.multiple_of(cid * N + s, 8), l)])
        _split_rows(sid, cp)

    wb(do_sh, dout_hbm)
    wb(di_sh, din_hbm)


@functools.partial(
    pl.kernel,
    out_type=[
        jax.ShapeDtypeStruct((NC * N, D), jnp.float32),
        jax.ShapeDtypeStruct((NC * N,), jnp.float32),
    ],
    mesh=_mesh,
    scratch_types=[
        pltpu.VMEM((NCHUNK, CH), jnp.int32),
        pltpu.VMEM((NCHUNK, CH), jnp.int32),
        pltpu.VMEM((CH, D), jnp.float32),
        pltpu.VMEM((CH,), jnp.float32),
        pltpu.VMEM((RB,), jnp.float32),
        pltpu.VMEM_SHARED((N, D), jnp.float32),
        pltpu.VMEM_SHARED((N,), jnp.float32),
        pltpu.VMEM_SHARED((N,), jnp.float32),
        pltpu.SemaphoreType.DMA,
        pltpu.SemaphoreType.DMA,
    ],
)
def _sc_edge_pass(src_hbm, dst_hbm, xw1_hbm, ni_hbm, agg_hbm, c_hbm,
                  sidx, didx, rows, vals, bounce, agg_sh, c_sh, ni_sh,
                  sem_r, sem_n):
    cid = lax.axis_index("c")
    sid = lax.axis_index("s")
    wid = cid * NS + sid
    _zero_rows(rows, CH, D)
    _zero_vec(bounce, RB)

    def zrows(s, l):
        for k in range(l // CH):
            pltpu.sync_copy(rows, agg_sh.at[pl.ds(s + k * CH, CH)])
    _split_rows(sid, zrows)
    _split_rows(sid, lambda s, l: pltpu.sync_copy(
        bounce.at[pl.ds(0, l)], c_sh.at[pl.ds(s, l)]))

    def stage_ni(s, l):
        pltpu.sync_copy(ni_hbm.at[pl.ds(s, l)], bounce.at[pl.ds(0, l)])
        pltpu.sync_copy(bounce.at[pl.ds(0, l)], ni_sh.at[pl.ds(s, l)])
    _split_rows(sid, stage_ni)

    pltpu.sync_copy(src_hbm.at[wid], sidx)
    pltpu.sync_copy(dst_hbm.at[wid], didx)
    plsc.subcore_barrier()

    def body(j, carry):
        gr = pltpu.async_copy(xw1_hbm.at[sidx.at[j]], rows, sem_r)
        gn = pltpu.async_copy(ni_sh.at[didx.at[j]], vals, sem_n)
        gr.wait()
        pltpu.sync_copy(rows, agg_sh.at[didx.at[j]], add=True)
        gn.wait()
        pltpu.sync_copy(vals, c_sh.at[sidx.at[j]], add=True)
        return carry

    lax.fori_loop(0, NCHUNK, body, 0)
    plsc.subcore_barrier()

    def wb_agg(s, l):
        for k in range(l // CH):
            pltpu.sync_copy(agg_sh.at[pl.ds(s + k * CH, CH)], rows)
            pltpu.sync_copy(
                rows,
                agg_hbm.at[pl.ds(pl.multiple_of(cid * N + s + k * CH, 8), CH)])
    _split_rows(sid, wb_agg)

    def wb_c(s, l):
        pltpu.sync_copy(c_sh.at[pl.ds(s, l)], bounce.at[pl.ds(0, l)])
        pltpu.sync_copy(bounce.at[pl.ds(0, l)],
                        c_hbm.at[pl.ds(pl.multiple_of(cid * N + s, 8), l)])
    _split_rows(sid, wb_c)


_BN = 2000
_G = N // _BN


def _tc_prep_body(dout_ref, din_ref, h_ref, w1_ref,
                  xw_ref, ni_ref, no_ref, hs_ref):
    i = pl.program_id(0)
    do = dout_ref[0] + dout_ref[1]
    di = din_ref[0] + din_ref[1]
    no = jnp.where(do > 0, lax.rsqrt(do), 0.0)
    ni = jnp.where(di > 0, lax.rsqrt(di), 0.0)
    hb = h_ref[...]
    xw_ref[...] = jnp.dot(hb * no, w1_ref[...],
                          preferred_element_type=jnp.float32)
    ni_ref[...] = ni
    no_ref[...] = no

    @pl.when(i == 0)
    def _init():
        hs_ref[...] = jnp.zeros_like(hs_ref)

    hs_ref[...] += jnp.sum(hb, axis=0, keepdims=True)


def _tc_prep(deg_out_p, deg_in_p, h, w1):
    return pl.pallas_call(
        _tc_prep_body,
        grid=(_G,),
        in_specs=[
            pl.BlockSpec((NC, _BN, 1), lambda i: (0, i, 0)),
            pl.BlockSpec((NC, _BN, 1), lambda i: (0, i, 0)),
            pl.BlockSpec((_BN, D), lambda i: (i, 0)),
            pl.BlockSpec((D, D), lambda i: (0, 0)),
        ],
        out_specs=[
            pl.BlockSpec((_BN, D), lambda i: (i, 0)),
            pl.BlockSpec((_BN, 1), lambda i: (i, 0)),
            pl.BlockSpec((_BN, 1), lambda i: (i, 0)),
            pl.BlockSpec((1, D), lambda i: (0, 0)),
        ],
        out_shape=[
            jax.ShapeDtypeStruct((N, D), jnp.float32),
            jax.ShapeDtypeStruct((N, 1), jnp.float32),
            jax.ShapeDtypeStruct((N, 1), jnp.float32),
            jax.ShapeDtypeStruct((1, D), jnp.float32),
        ],
    )(deg_out_p, deg_in_p, h, w1)


def _tc_final_body(agg_ref, c_ref, ni_ref, no_ref, b1_ref, w2_ref, b2_ref,
                   wlin_ref, blin_ref, hs_ref, out_ref, acc_ref):
    i = pl.program_id(0)

    @pl.when(i == 0)
    def _init():
        acc_ref[...] = jnp.zeros_like(acc_ref)

    agg = agg_ref[0] + agg_ref[1]
    x1 = jnp.maximum(agg * ni_ref[...] + b1_ref[...], 0.0)
    c = c_ref[0] + c_ref[1]
    w = no_ref[...] * c
    acc_ref[...] += jnp.sum(x1 * w, axis=0, keepdims=True)

    @pl.when(i == pl.num_programs(0) - 1)
    def _fin():
        v = acc_ref[...]
        out_ref[...] = (
            jnp.dot(v, w2_ref[...], preferred_element_type=jnp.float32) / N
            + b2_ref[...]
            + jnp.dot(hs_ref[...] / N, wlin_ref[...],
                      preferred_element_type=jnp.float32)
            + blin_ref[...])


def _tc_final(agg_p, c_p, ni1, no1, b1, w2, b2, wlin, blin, hsum):
    return pl.pallas_call(
        _tc_final_body,
        grid=(_G,),
        in_specs=[
            pl.BlockSpec((NC, _BN, D), lambda i: (0, i, 0)),
            pl.BlockSpec((NC, _BN, 1), lambda i: (0, i, 0)),
            pl.BlockSpec((_BN, 1), lambda i: (i, 0)),
            pl.BlockSpec((_BN, 1), lambda i: (i, 0)),
            pl.BlockSpec((1, D), lambda i: (0, 0)),
            pl.BlockSpec((D, D), lambda i: (0, 0)),
            pl.BlockSpec((1, D), lambda i: (0, 0)),
            pl.BlockSpec((D, D), lambda i: (0, 0)),
            pl.BlockSpec((1, D), lambda i: (0, 0)),
            pl.BlockSpec((1, D), lambda i: (0, 0)),
        ],
        out_specs=pl.BlockSpec((1, D), lambda i: (0, 0)),
        out_shape=jax.ShapeDtypeStruct((1, D), jnp.float32),
        scratch_shapes=[pltpu.VMEM((1, D), jnp.float32)],
    )(agg_p, c_p, ni1, no1, b1, w2, b2, wlin, blin, hsum)


def kernel(h, edge_index, W1, b1, W2, b2, Wlin, blin):
    src2d = edge_index[0].reshape(NW, NCHUNK, CH)
    dst2d = edge_index[1].reshape(NW, NCHUNK, CH)

    dout_p, din_p = _sc_degrees(src2d, dst2d)
    xw1, ni1, no1, hsum = _tc_prep(
        dout_p.reshape(NC, N, 1), din_p.reshape(NC, N, 1), h, W1)

    agg_p, c_p = _sc_edge_pass(src2d, dst2d, xw1, ni1.reshape(N))

    return _tc_final(agg_p.reshape(NC, N, D), c_p.reshape(NC, N, 1),
                     ni1, no1, b1.reshape(1, D), W2,
                     b2.reshape(1, D), Wlin, blin.reshape(1, D), hsum)


# speedup vs baseline: 16.7439x; 1.4568x over previous
import functools

import jax
import jax.numpy as jnp
from jax import lax
from jax.experimental import pallas as pl
from jax.experimental.pallas import tpu as pltpu
from jax.experimental.pallas import tpu_sc as plsc

N = 10000
E = 320000
D = 128
NC = 2
NS = 16
NW = NC * NS
EPW = E // NW
CH = 80
NCHUNK = EPW // CH
RB = 640
RBL = N - (NS - 1) * RB

_mesh = plsc.VectorSubcoreMesh(
    core_axis_name="c", subcore_axis_name="s", num_cores=NC, num_subcores=NS)


def _zero_vec(ref, n):
    for q in range(n // 16):
        ref[pl.ds(q * 16, 16)] = jnp.zeros((16,), jnp.float32)


def _zero_rows(ref, r, w):
    for i in range(r):
        for q in range(w // 16):
            ref[i, pl.ds(q * 16, 16)] = jnp.zeros((16,), jnp.float32)


def _split_rows(sid, fn):
    @pl.when(sid < NS - 1)
    def _main():
        fn(pl.multiple_of(sid * RB, 8), RB)

    @pl.when(sid == NS - 1)
    def _last():
        fn((NS - 1) * RB, RBL)


@functools.partial(
    pl.kernel,
    out_type=[
        jax.ShapeDtypeStruct((NC * N,), jnp.float32),
        jax.ShapeDtypeStruct((NC * N,), jnp.float32),
    ],
    mesh=_mesh,
    compiler_params=pltpu.CompilerParams(use_tc_tiling_on_sc=False),
    scratch_types=[
        pltpu.VMEM((NCHUNK, CH), jnp.int32),
        pltpu.VMEM((NCHUNK, CH), jnp.int32),
        pltpu.VMEM((CH,), jnp.float32),
        pltpu.VMEM((RB,), jnp.float32),
        pltpu.VMEM_SHARED((N,), jnp.float32),
        pltpu.VMEM_SHARED((N,), jnp.float32),
        pltpu.SemaphoreType.DMA,
    ],
)
def _sc_degrees(src_hbm, dst_hbm, dout_hbm, din_hbm,
                sidx, didx, ones_v, bounce, do_sh, di_sh, sem_a):
    cid = lax.axis_index("c")
    sid = lax.axis_index("s")
    wid = cid * NS + sid
    for q in range(CH // 16):
        ones_v[pl.ds(q * 16, 16)] = jnp.ones((16,), jnp.float32)
    _zero_vec(bounce, RB)
    _split_rows(sid, lambda s, l: pltpu.sync_copy(
        bounce.at[pl.ds(0, l)], do_sh.at[pl.ds(s, l)]))
    _split_rows(sid, lambda s, l: pltpu.sync_copy(
        bounce.at[pl.ds(0, l)], di_sh.at[pl.ds(s, l)]))
    pltpu.sync_copy(src_hbm.at[wid], sidx)
    pltpu.sync_copy(dst_hbm.at[wid], didx)
    plsc.subcore_barrier()

    RING = 4

    def fire1(j):
        pltpu.async_copy(ones_v, do_sh.at[sidx.at[j]], sem_a, add=True)
        pltpu.async_copy(ones_v, di_sh.at[didx.at[j]], sem_a, add=True)

    def wait1(j):
        pltpu.make_async_copy(ones_v, do_sh.at[sidx.at[j]], sem_a).wait()
        pltpu.make_async_copy(ones_v, di_sh.at[didx.at[j]], sem_a).wait()

    def body(j, carry):
        fire1(j)

        @pl.when(j >= RING)
        def _drain():
            wait1(j - RING)

        return carry

    lax.fori_loop(0, NCHUNK, body, 0)
    for t in range(RING):
        wait1(NCHUNK - RING + t)
    plsc.subcore_barrier()

    def wb(sh, out):
        def cp(s, l):
            pltpu.sync_copy(sh.at[pl.ds(s, l)], bounce.at[pl.ds(0, l)])
            pltpu.sync_copy(bounce.at[pl.ds(0, l)],
                            out.at[pl.ds(pl.multiple_of(cid * N + s, 8), l)])
        _split_rows(sid, cp)

    wb(do_sh, dout_hbm)
    wb(di_sh, din_hbm)


@functools.partial(
    pl.kernel,
    out_type=[
        jax.ShapeDtypeStruct((NC * N, D), jnp.float32),
        jax.ShapeDtypeStruct((NC * N,), jnp.float32),
    ],
    mesh=_mesh,
    compiler_params=pltpu.CompilerParams(use_tc_tiling_on_sc=False),
    scratch_types=[
        pltpu.VMEM((63, CH), jnp.int32),
        pltpu.VMEM((63, CH), jnp.int32),
        pltpu.VMEM((CH, D), jnp.float32),
        pltpu.VMEM((CH, D), jnp.float32),
        pltpu.VMEM((CH,), jnp.float32),
        pltpu.VMEM((CH,), jnp.float32),
        pltpu.VMEM((RB,), jnp.float32),
        pltpu.VMEM_SHARED((N, D), jnp.float32),
        pltpu.VMEM_SHARED((N,), jnp.float32),
        pltpu.VMEM_SHARED((N,), jnp.float32),
        pltpu.SemaphoreType.DMA,
        pltpu.SemaphoreType.DMA,
        pltpu.SemaphoreType.DMA,
        pltpu.SemaphoreType.DMA,
    ],
)
def _sc_edge_pass(src_hbm, dst_hbm, xw1_hbm, ni_hbm, agg_hbm, c_hbm,
                  sidx, didx, rows, rows1, vals, vals1, bounce,
                  agg_sh, c_sh, ni_sh, sem_r, sem_n, sem_r1, sem_n1):
    cid = lax.axis_index("c")
    sid = lax.axis_index("s")
    wid = cid * NS + sid
    _zero_rows(rows, CH, D)
    _zero_vec(bounce, RB)

    def zrows(s, l):
        for k in range(l // CH):
            pltpu.sync_copy(rows, agg_sh.at[pl.ds(s + k * CH, CH)])
    _split_rows(sid, zrows)
    _split_rows(sid, lambda s, l: pltpu.sync_copy(
        bounce.at[pl.ds(0, l)], c_sh.at[pl.ds(s, l)]))

    def stage_ni(s, l):
        pltpu.sync_copy(ni_hbm.at[pl.ds(s, l)], bounce.at[pl.ds(0, l)])
        pltpu.sync_copy(bounce.at[pl.ds(0, l)], ni_sh.at[pl.ds(s, l)])
    _split_rows(sid, stage_ni)

    plsc.subcore_barrier()

    bufs = ((rows, vals, sem_r, sem_n), (rows1, vals1, sem_r1, sem_n1))

    def fire(c, b):
        rb, vb, sr, sn = bufs[b]
        pltpu.async_copy(xw1_hbm.at[sidx.at[c]], rb, sr)
        pltpu.async_copy(ni_sh.at[didx.at[c]], vb, sn)

    def drain_scatter(c, b):
        rb, vb, sr, sn = bufs[b]
        pltpu.make_async_copy(xw1_hbm.at[pl.ds(0, CH)], rb, sr).wait()
        pltpu.sync_copy(rb, agg_sh.at[didx.at[c]], add=True)
        pltpu.make_async_copy(ni_hbm.at[pl.ds(0, CH)], vb, sn).wait()
        pltpu.sync_copy(vb, c_sh.at[sidx.at[c]], add=True)

    for h0, hn in ((0, 63), (63, NCHUNK - 63)):
        pltpu.sync_copy(src_hbm.at[wid, pl.ds(h0, hn)], sidx.at[pl.ds(0, hn)])
        pltpu.sync_copy(dst_hbm.at[wid, pl.ds(h0, hn)], didx.at[pl.ds(0, hn)])
        fire(0, 0)
        np_ = (hn - 1) // 2

        def body(j, carry):
            c0 = 2 * j
            fire(c0 + 1, 1)
            drain_scatter(c0, 0)
            fire(c0 + 2, 0)
            drain_scatter(c0 + 1, 1)
            return carry

        lax.fori_loop(0, np_, body, 0)
        drain_scatter(2 * np_, 0)
        if hn - 2 * np_ == 2:
            fire(2 * np_ + 1, 1)
            drain_scatter(2 * np_ + 1, 1)
    plsc.subcore_barrier()

    def wb_agg(s, l):
        for k in range(l // CH):
            pltpu.sync_copy(agg_sh.at[pl.ds(s + k * CH, CH)], rows)
            pltpu.sync_copy(
                rows,
                agg_hbm.at[pl.ds(pl.multiple_of(cid * N + s + k * CH, 8), CH)])
    _split_rows(sid, wb_agg)

    def wb_c(s, l):
        pltpu.sync_copy(c_sh.at[pl.ds(s, l)], bounce.at[pl.ds(0, l)])
        pltpu.sync_copy(bounce.at[pl.ds(0, l)],
                        c_hbm.at[pl.ds(pl.multiple_of(cid * N + s, 8), l)])
    _split_rows(sid, wb_c)


_BN = 10000
_G = N // _BN


def _tc_prep_body(dout_ref, din_ref, h_ref, w1_ref,
                  xw_ref, ni_ref, no_ref, hs_ref):
    i = pl.program_id(0)
    do = dout_ref[0] + dout_ref[1]
    di = din_ref[0] + din_ref[1]
    no = jnp.where(do > 0, lax.rsqrt(do), 0.0)
    ni = jnp.where(di > 0, lax.rsqrt(di), 0.0)
    hb = h_ref[...]
    xw_ref[...] = jnp.dot(hb * no, w1_ref[...],
                          preferred_element_type=jnp.float32)
    ni_ref[...] = ni
    no_ref[...] = no

    @pl.when(i == 0)
    def _init():
        hs_ref[...] = jnp.zeros_like(hs_ref)

    hs_ref[...] += jnp.sum(hb, axis=0, keepdims=True)


def _tc_prep(deg_out_p, deg_in_p, h, w1):
    return pl.pallas_call(
        _tc_prep_body,
        grid=(_G,),
        in_specs=[
            pl.BlockSpec((NC, _BN, 1), lambda i: (0, i, 0)),
            pl.BlockSpec((NC, _BN, 1), lambda i: (0, i, 0)),
            pl.BlockSpec((_BN, D), lambda i: (i, 0)),
            pl.BlockSpec((D, D), lambda i: (0, 0)),
        ],
        out_specs=[
            pl.BlockSpec((_BN, D), lambda i: (i, 0)),
            pl.BlockSpec((_BN, 1), lambda i: (i, 0)),
            pl.BlockSpec((_BN, 1), lambda i: (i, 0)),
            pl.BlockSpec((1, D), lambda i: (0, 0)),
        ],
        out_shape=[
            jax.ShapeDtypeStruct((N, D), jnp.float32),
            jax.ShapeDtypeStruct((N, 1), jnp.float32),
            jax.ShapeDtypeStruct((N, 1), jnp.float32),
            jax.ShapeDtypeStruct((1, D), jnp.float32),
        ],
    )(deg_out_p, deg_in_p, h, w1)


def _tc_final_body(agg_ref, c_ref, ni_ref, no_ref, b1_ref, w2_ref, b2_ref,
                   wlin_ref, blin_ref, hs_ref, out_ref, acc_ref):
    i = pl.program_id(0)

    @pl.when(i == 0)
    def _init():
        acc_ref[...] = jnp.zeros_like(acc_ref)

    agg = agg_ref[0] + agg_ref[1]
    x1 = jnp.maximum(agg * ni_ref[...] + b1_ref[...], 0.0)
    c = c_ref[0] + c_ref[1]
    w = no_ref[...] * c
    acc_ref[...] += jnp.sum(x1 * w, axis=0, keepdims=True)

    @pl.when(i == pl.num_programs(0) - 1)
    def _fin():
        v = acc_ref[...]
        out_ref[...] = (
            jnp.dot(v, w2_ref[...], preferred_element_type=jnp.float32) / N
            + b2_ref[...]
            + jnp.dot(hs_ref[...] / N, wlin_ref[...],
                      preferred_element_type=jnp.float32)
            + blin_ref[...])


def _tc_final(agg_p, c_p, ni1, no1, b1, w2, b2, wlin, blin, hsum):
    return pl.pallas_call(
        _tc_final_body,
        grid=(_G,),
        in_specs=[
            pl.BlockSpec((NC, _BN, D), lambda i: (0, i, 0)),
            pl.BlockSpec((NC, _BN, 1), lambda i: (0, i, 0)),
            pl.BlockSpec((_BN, 1), lambda i: (i, 0)),
            pl.BlockSpec((_BN, 1), lambda i: (i, 0)),
            pl.BlockSpec((1, D), lambda i: (0, 0)),
            pl.BlockSpec((D, D), lambda i: (0, 0)),
            pl.BlockSpec((1, D), lambda i: (0, 0)),
            pl.BlockSpec((D, D), lambda i: (0, 0)),
            pl.BlockSpec((1, D), lambda i: (0, 0)),
            pl.BlockSpec((1, D), lambda i: (0, 0)),
        ],
        out_specs=pl.BlockSpec((1, D), lambda i: (0, 0)),
        out_shape=jax.ShapeDtypeStruct((1, D), jnp.float32),
        scratch_shapes=[pltpu.VMEM((1, D), jnp.float32)],
    )(agg_p, c_p, ni1, no1, b1, w2, b2, wlin, blin, hsum)


def kernel(h, edge_index, W1, b1, W2, b2, Wlin, blin):
    src2d = edge_index[0].reshape(NW, NCHUNK, CH)
    dst2d = edge_index[1].reshape(NW, NCHUNK, CH)

    dout_p, din_p = _sc_degrees(src2d, dst2d)
    xw1, ni1, no1, hsum = _tc_prep(
        dout_p.reshape(NC, N, 1), din_p.reshape(NC, N, 1), h, W1)

    agg_p, c_p = _sc_edge_pass(src2d, dst2d, xw1, ni1.reshape(N))

    return _tc_final(agg_p.reshape(NC, N, D), c_p.reshape(NC, N, 1),
                     ni1, no1, b1.reshape(1, D), W2,
                     b2.reshape(1, D), Wlin, blin.reshape(1, D), hsum)
